# Initial kernel scaffold; baseline (speedup 1.0000x reference)
import functools

import jax
import jax.numpy as jnp
from jax import lax
from jax.experimental import pallas as pl
from jax.experimental.pallas import tpu as pltpu
from jax.experimental.pallas import tpu_sc as plsc

NNODES = 10000
DIN = 128
HID = 256
NCLS = 20
NEDGE = 320000

NTILE = 16
NPAD = 10240
RPT = NPAD // NTILE
GSZ = 128
GROUPS = 160
EPAD = NTILE * GROUPS * GSZ
GPC = GROUPS // 2
NB = 4

_MESH = plsc.VectorSubcoreMesh(core_axis_name="c", subcore_axis_name="s")


def _deg_body(dst_w, zeros_h, ones_h, pdeg, acc, didx, ones_v, ssem):
    c = lax.axis_index("c")
    s = lax.axis_index("s")
    row0 = s * RPT
    pltpu.sync_copy(zeros_h.at[pl.ds(row0, RPT)], acc.at[pl.ds(row0, RPT)])
    pltpu.sync_copy(ones_h, ones_v)
    pltpu.sync_copy(dst_w.at[s, pl.ds(c * GPC, GPC)], didx)
    plsc.subcore_barrier()

    def outer(i, carry):
        descs = []
        for b in range(NB):
            g = i * NB + b
            descs.append(
                pltpu.async_copy(ones_v, acc.at[didx.at[g]], ssem, add=True))
        for d in descs:
            d.wait()
        return carry

    lax.fori_loop(0, GPC // NB, outer, 0)
    plsc.subcore_barrier()
    pltpu.sync_copy(acc.at[pl.ds(row0, RPT)], pdeg.at[c, pl.ds(row0, RPT)])


_deg_call = pl.kernel(
    _deg_body,
    out_type=jax.ShapeDtypeStruct((2, NPAD, 16), jnp.float32),
    mesh=_MESH,
    scratch_types=[
        pltpu.VMEM_SHARED((NPAD, 16), jnp.float32),
        pltpu.VMEM((GPC, GSZ), jnp.int32),
        pltpu.VMEM((GSZ, 16), jnp.float32),
        pltpu.SemaphoreType.DMA,
    ],
    name="sc_degree_count",
)


def _agg_body(hs, src_w, dst_w, seg, acc, sidx, didx, r0, r1, r2, r3, gsem,
              ssem):
    c = lax.axis_index("c")
    s = lax.axis_index("s")
    rows = [r0, r1, r2, r3]
    row0 = s * RPT
    pltpu.sync_copy(hs.at[pl.ds(c * NPAD + row0, RPT)], acc.at[pl.ds(row0, RPT)])
    pltpu.sync_copy(src_w.at[c, s], sidx)
    pltpu.sync_copy(dst_w.at[s], didx)
    plsc.subcore_barrier()

    def outer(i, carry):
        gdescs = []
        for b in range(NB):
            g = i * NB + b
            gdescs.append(pltpu.async_copy(hs.at[sidx.at[g]], rows[b], gsem))
        sdescs = []
        for b in range(NB):
            g = i * NB + b
            gdescs[b].wait()
            sdescs.append(
                pltpu.async_copy(rows[b], acc.at[didx.at[g]], ssem, add=True))
        for d in sdescs:
            d.wait()
        return carry

    lax.fori_loop(0, GROUPS // NB, outer, 0)
    plsc.subcore_barrier()
    pltpu.sync_copy(acc.at[pl.ds(row0, RPT)], seg.at[c, pl.ds(row0, RPT)])


_agg_call = pl.kernel(
    _agg_body,
    out_type=jax.ShapeDtypeStruct((2, NPAD, 128), jnp.float32),
    mesh=_MESH,
    scratch_types=[
        pltpu.VMEM_SHARED((NPAD, 128), jnp.float32),
        pltpu.VMEM((GROUPS, GSZ), jnp.int32),
        pltpu.VMEM((GROUPS, GSZ), jnp.int32),
        pltpu.VMEM((GSZ, 128), jnp.float32),
        pltpu.VMEM((GSZ, 128), jnp.float32),
        pltpu.VMEM((GSZ, 128), jnp.float32),
        pltpu.VMEM((GSZ, 128), jnp.float32),
        pltpu.SemaphoreType.DMA,
        pltpu.SemaphoreType.DMA,
    ],
    name="sc_edge_aggregate",
)


RBLK = 512
MGRID = NPAD // RBLK


def _k1_body(x_ref, pd_ref, w_ref, hs_ref, dinv_ref):
    pd = pd_ref[...]
    cnt = pd[0, :, 0:1] + pd[1, :, 0:1]
    dv = lax.rsqrt(1.0 + cnt)
    dinv_ref[...] = dv
    hs_ref[...] = dv * jnp.dot(x_ref[...], w_ref[...],
                               preferred_element_type=jnp.float32)


def _k1_call(xp, pdeg, w1):
    return pl.pallas_call(
        _k1_body,
        grid=(MGRID, 2),
        in_specs=[
            pl.BlockSpec((RBLK, DIN), lambda i, h: (i, 0)),
            pl.BlockSpec((2, RBLK, 16), lambda i, h: (0, i, 0)),
            pl.BlockSpec((DIN, 128), lambda i, h: (0, h)),
        ],
        out_specs=[
            pl.BlockSpec((RBLK, 128), lambda i, h: (h * MGRID + i, 0)),
            pl.BlockSpec((RBLK, 1), lambda i, h: (i, 0)),
        ],
        out_shape=[
            jax.ShapeDtypeStruct((2 * NPAD, 128), jnp.float32),
            jax.ShapeDtypeStruct((NPAD, 1), jnp.float32),
        ],
    )(xp, pdeg, w1)


def _mid_body(seg_ref, dinv_ref, al_ref, be_ref, w_ref, hs_ref):
    sg = seg_ref[...]
    dv = dinv_ref[...]
    z = jnp.concatenate([sg[0], sg[1]], axis=1)
    z = jnp.maximum(dv * z * al_ref[...] + be_ref[...], 0.0)
    hs_ref[...] = dv * jnp.dot(z, w_ref[...],
                               preferred_element_type=jnp.float32)


def _mid_call(seg, dinv, alpha, beta, w):
    return pl.pallas_call(
        _mid_body,
        grid=(MGRID, 2),
        in_specs=[
            pl.BlockSpec((2, RBLK, 128), lambda i, h: (0, i, 0)),
            pl.BlockSpec((RBLK, 1), lambda i, h: (i, 0)),
            pl.BlockSpec((1, HID), lambda i, h: (0, 0)),
            pl.BlockSpec((1, HID), lambda i, h: (0, 0)),
            pl.BlockSpec((HID, 128), lambda i, h: (0, h)),
        ],
        out_specs=pl.BlockSpec((RBLK, 128), lambda i, h: (h * MGRID + i, 0)),
        out_shape=jax.ShapeDtypeStruct((2 * NPAD, 128), jnp.float32),
    )(seg, dinv, alpha, beta, w)


def _head_body(seg_ref, dinv_ref, b3_ref, wc1_ref, bc1_ref, wc2_ref, bc2_ref,
               out_ref):
    sg = seg_ref[...]
    dv = dinv_ref[...]
    z3 = dv * jnp.concatenate([sg[0], sg[1]], axis=1) + b3_ref[...]
    t = jnp.maximum(
        jnp.dot(z3, wc1_ref[...], preferred_element_type=jnp.float32)
        + bc1_ref[...], 0.0)
    out_ref[...] = (jnp.dot(t, wc2_ref[...],
                            preferred_element_type=jnp.float32) + bc2_ref[...])


def _head_call(seg, dinv, b3, wc1, bc1, wc2p, bc2p):
    return pl.pallas_call(
        _head_body,
        grid=(MGRID,),
        in_specs=[
            pl.BlockSpec((2, RBLK, 128), lambda i: (0, i, 0)),
            pl.BlockSpec((RBLK, 1), lambda i: (i, 0)),
            pl.BlockSpec((1, HID), lambda i: (0, 0)),
            pl.BlockSpec((HID, 128), lambda i: (0, 0)),
            pl.BlockSpec((1, 128), lambda i: (0, 0)),
            pl.BlockSpec((128, 128), lambda i: (0, 0)),
            pl.BlockSpec((1, 128), lambda i: (0, 0)),
        ],
        out_specs=pl.BlockSpec((RBLK, 128), lambda i: (i, 0)),
        out_shape=jax.ShapeDtypeStruct((NPAD, 128), jnp.float32),
    )(seg, dinv, b3, wc1, bc1, wc2p, bc2p)


def kernel(x, edge_index, W1, b1, g1, be1, rm1, rv1, W2, b2, g2, be2, rm2,
           rv2, W3, b3, Wc1, bc1, Wc2, bc2):
    f32 = jnp.float32
    src = edge_index[0]
    dst = edge_index[1]

    xp = jnp.concatenate([x, jnp.zeros((NPAD - NNODES, DIN), f32)], axis=0)
    pad = jnp.full((EPAD - NEDGE,), NPAD - 1, jnp.int32)
    src_w = jnp.concatenate([src, pad]).reshape(NTILE, GROUPS, GSZ)
    dst_w = jnp.concatenate([dst, pad]).reshape(NTILE, GROUPS, GSZ)
    src_w2 = jnp.stack([src_w, src_w + NPAD])

    zeros_h = jnp.zeros((NPAD, 16), f32)
    ones_h = jnp.ones((GSZ, 16), f32)
    pdeg = _deg_call(dst_w, zeros_h, ones_h)

    hs1, dinv = _k1_call(xp, pdeg, W1)
    seg1 = _agg_call(hs1, src_w2, dst_w)

    a1 = g1 * lax.rsqrt(rv1 + 1e-5)
    al1 = a1.reshape(1, HID)
    bt1 = (a1 * b1 + be1 - rm1 * a1).reshape(1, HID)
    hs2 = _mid_call(seg1, dinv, al1, bt1, W2)
    seg2 = _agg_call(hs2, src_w2, dst_w)

    a2 = g2 * lax.rsqrt(rv2 + 1e-5)
    al2 = a2.reshape(1, HID)
    bt2 = (a2 * b2 + be2 - rm2 * a2).reshape(1, HID)
    hs3 = _mid_call(seg2, dinv, al2, bt2, W3)
    seg3 = _agg_call(hs3, src_w2, dst_w)

    wc2p = jnp.zeros((128, 128), f32).at[:, :NCLS].set(Wc2)
    bc2p = jnp.zeros((1, 128), f32).at[0, :NCLS].set(bc2)
    out = _head_call(seg3, dinv, b3.reshape(1, HID), Wc1, bc1.reshape(1, 128),
                     wc2p, bc2p)
    return out[:NNODES, :NCLS]


# speedup vs baseline: 7.2334x; 7.2334x over previous
import functools

import jax
import jax.numpy as jnp
from jax import lax
from jax.experimental import pallas as pl
from jax.experimental.pallas import tpu as pltpu
from jax.experimental.pallas import tpu_sc as plsc

NNODES = 10000
DIN = 128
HID = 256
NCLS = 20
NEDGE = 320000

NTILE = 16
NPAD = 10240
RPT = NPAD // NTILE
GSZ = 128
GROUPS = 160
EPAD = NTILE * GROUPS * GSZ
GPC = GROUPS // 2
NB = 2

@functools.cache
def _mesh():
    return plsc.VectorSubcoreMesh(core_axis_name="c", subcore_axis_name="s",
                                  num_cores=2, num_subcores=NTILE)


def _deg_body(dst_w, zeros_h, ones_h, pdeg, acc, didx, ones_v, ssem):
    c = lax.axis_index("c")
    s = lax.axis_index("s")
    row0 = s * RPT
    pltpu.sync_copy(zeros_h.at[pl.ds(row0, RPT)], acc.at[pl.ds(row0, RPT)])
    pltpu.sync_copy(ones_h, ones_v)
    pltpu.sync_copy(dst_w.at[s, pl.ds(c * GPC, GPC)], didx)
    plsc.subcore_barrier()

    def outer(i, carry):
        descs = []
        for b in range(NB):
            g = i * NB + b
            descs.append(
                pltpu.async_copy(ones_v, acc.at[didx.at[g]], ssem, add=True))
        for d in descs:
            d.wait()
        return carry

    lax.fori_loop(0, GPC // NB, outer, 0)
    plsc.subcore_barrier()
    pltpu.sync_copy(acc.at[pl.ds(row0, RPT)], pdeg.at[c, pl.ds(row0, RPT)])


@functools.cache
def _deg_call():
    return pl.kernel(
        _deg_body,
        out_type=jax.ShapeDtypeStruct((2, NPAD, 128), jnp.float32),
        mesh=_mesh(),
        scratch_types=[
            pltpu.VMEM_SHARED((NPAD, 128), jnp.float32),
            pltpu.VMEM((GPC, GSZ), jnp.int32),
            pltpu.VMEM((GSZ, 128), jnp.float32),
            pltpu.SemaphoreType.DMA,
        ],
        name="sc_degree_count",
    )


def _agg_body(hs, src_w, dst_w, seg, acc, r0, r1, sa, da, sb, db, isem, g0sem,
              g1sem, s0sem, s1sem):
    c = lax.axis_index("c")
    s = lax.axis_index("s")
    rows = (r0, r1)
    gsems = (g0sem, g1sem)
    ssems = (s0sem, s1sem)
    row0 = s * RPT
    pltpu.sync_copy(hs.at[pl.ds(c * NPAD + row0, RPT)], acc.at[pl.ds(row0, RPT)])
    pltpu.sync_copy(src_w.at[c, s, pl.ds(0, NB)], sa)
    pltpu.sync_copy(dst_w.at[s, pl.ds(0, NB)], da)
    plsc.subcore_barrier()

    def step(jj, cs, cd, ns, nd):
        pi1 = pltpu.async_copy(src_w.at[c, s, pl.ds((jj + 1) * NB, NB)], ns,
                               isem)
        pi2 = pltpu.async_copy(dst_w.at[s, pl.ds((jj + 1) * NB, NB)], nd, isem)
        gd = [pltpu.async_copy(hs.at[cs.at[b]], rows[b], gsems[b])
              for b in range(NB)]
        sd = []
        for b in range(NB):
            gd[b].wait()
            sd.append(
                pltpu.async_copy(rows[b], acc.at[cd.at[b]], ssems[b],
                                 add=True))
        for d in sd:
            d.wait()
        pi1.wait()
        pi2.wait()

    def outer(k, carry):
        step(2 * k, sa, da, sb, db)
        step(2 * k + 1, sb, db, sa, da)
        return carry

    lax.fori_loop(0, GROUPS // NB // 2, outer, 0)
    plsc.subcore_barrier()
    pltpu.sync_copy(acc.at[pl.ds(row0, RPT)], seg.at[c, pl.ds(row0, RPT)])


@functools.cache
def _agg_call():
    return pl.kernel(
        _agg_body,
        out_type=jax.ShapeDtypeStruct((2, NPAD, 128), jnp.float32),
        mesh=_mesh(),
        scratch_types=[
            pltpu.VMEM_SHARED((NPAD, 128), jnp.float32),
            pltpu.VMEM((GSZ, 128), jnp.float32),
            pltpu.VMEM((GSZ, 128), jnp.float32),
            pltpu.VMEM((NB, GSZ), jnp.int32),
            pltpu.VMEM((NB, GSZ), jnp.int32),
            pltpu.VMEM((NB, GSZ), jnp.int32),
            pltpu.VMEM((NB, GSZ), jnp.int32),
            pltpu.SemaphoreType.DMA,
            pltpu.SemaphoreType.DMA,
            pltpu.SemaphoreType.DMA,
            pltpu.SemaphoreType.DMA,
            pltpu.SemaphoreType.DMA,
        ],
        name="sc_edge_aggregate",
    )


RBLK = 512
MGRID = NPAD // RBLK


def _k1_body(x_ref, pd_ref, w_ref, hs_ref, dinv_ref):
    pd = pd_ref[...]
    cnt = pd[0, :, 0:1] + pd[1, :, 0:1]
    dv = lax.rsqrt(1.0 + cnt)
    dinv_ref[...] = dv
    hs_ref[...] = dv * jnp.dot(x_ref[...], w_ref[...],
                               preferred_element_type=jnp.float32)


def _k1_call(xp, pdeg, w1):
    return pl.pallas_call(
        _k1_body,
        grid=(MGRID, 2),
        in_specs=[
            pl.BlockSpec((RBLK, DIN), lambda i, h: (i, 0)),
            pl.BlockSpec((2, RBLK, 128), lambda i, h: (0, i, 0)),
            pl.BlockSpec((DIN, 128), lambda i, h: (0, h)),
        ],
        out_specs=[
            pl.BlockSpec((RBLK, 128), lambda i, h: (h * MGRID + i, 0)),
            pl.BlockSpec((RBLK, 1), lambda i, h: (i, 0)),
        ],
        out_shape=[
            jax.ShapeDtypeStruct((2 * NPAD, 128), jnp.float32),
            jax.ShapeDtypeStruct((NPAD, 1), jnp.float32),
        ],
    )(xp, pdeg, w1)


def _mid_body(seg_ref, dinv_ref, al_ref, be_ref, w_ref, hs_ref):
    sg = seg_ref[...]
    dv = dinv_ref[...]
    z = jnp.concatenate([sg[0], sg[1]], axis=1)
    z = jnp.maximum(dv * z * al_ref[...] + be_ref[...], 0.0)
    hs_ref[...] = dv * jnp.dot(z, w_ref[...],
                               preferred_element_type=jnp.float32)


def _mid_call(seg, dinv, alpha, beta, w):
    return pl.pallas_call(
        _mid_body,
        grid=(MGRID, 2),
        in_specs=[
            pl.BlockSpec((2, RBLK, 128), lambda i, h: (0, i, 0)),
            pl.BlockSpec((RBLK, 1), lambda i, h: (i, 0)),
            pl.BlockSpec((1, HID), lambda i, h: (0, 0)),
            pl.BlockSpec((1, HID), lambda i, h: (0, 0)),
            pl.BlockSpec((HID, 128), lambda i, h: (0, h)),
        ],
        out_specs=pl.BlockSpec((RBLK, 128), lambda i, h: (h * MGRID + i, 0)),
        out_shape=jax.ShapeDtypeStruct((2 * NPAD, 128), jnp.float32),
    )(seg, dinv, alpha, beta, w)


def _head_body(seg_ref, dinv_ref, b3_ref, wc1_ref, bc1_ref, wc2_ref, bc2_ref,
               out_ref):
    sg = seg_ref[...]
    dv = dinv_ref[...]
    z3 = dv * jnp.concatenate([sg[0], sg[1]], axis=1) + b3_ref[...]
    t = jnp.maximum(
        jnp.dot(z3, wc1_ref[...], preferred_element_type=jnp.float32)
        + bc1_ref[...], 0.0)
    out_ref[...] = (jnp.dot(t, wc2_ref[...],
                            preferred_element_type=jnp.float32) + bc2_ref[...])


def _head_call(seg, dinv, b3, wc1, bc1, wc2p, bc2p):
    return pl.pallas_call(
        _head_body,
        grid=(MGRID,),
        in_specs=[
            pl.BlockSpec((2, RBLK, 128), lambda i: (0, i, 0)),
            pl.BlockSpec((RBLK, 1), lambda i: (i, 0)),
            pl.BlockSpec((1, HID), lambda i: (0, 0)),
            pl.BlockSpec((HID, 128), lambda i: (0, 0)),
            pl.BlockSpec((1, 128), lambda i: (0, 0)),
            pl.BlockSpec((128, 128), lambda i: (0, 0)),
            pl.BlockSpec((1, 128), lambda i: (0, 0)),
        ],
        out_specs=pl.BlockSpec((RBLK, 128), lambda i: (i, 0)),
        out_shape=jax.ShapeDtypeStruct((NPAD, 128), jnp.float32),
    )(seg, dinv, b3, wc1, bc1, wc2p, bc2p)


def kernel(x, edge_index, W1, b1, g1, be1, rm1, rv1, W2, b2, g2, be2, rm2,
           rv2, W3, b3, Wc1, bc1, Wc2, bc2):
    f32 = jnp.float32
    src = edge_index[0]
    dst = edge_index[1]

    xp = jnp.concatenate([x, jnp.zeros((NPAD - NNODES, DIN), f32)], axis=0)
    pad = jnp.full((EPAD - NEDGE,), NPAD - 1, jnp.int32)
    xpad = jnp.full((NTILE, NB, GSZ), NPAD - 1, jnp.int32)
    src_w = jnp.concatenate(
        [jnp.concatenate([src, pad]).reshape(NTILE, GROUPS, GSZ), xpad], axis=1)
    dst_w = jnp.concatenate(
        [jnp.concatenate([dst, pad]).reshape(NTILE, GROUPS, GSZ), xpad], axis=1)
    src_w2 = jnp.stack([src_w, src_w + NPAD])

    zeros_h = jnp.zeros((NPAD, 128), f32)
    ones_h = jnp.ones((GSZ, 128), f32)
    pdeg = _deg_call()(dst_w, zeros_h, ones_h)

    hs1, dinv = _k1_call(xp, pdeg, W1)
    seg1 = _agg_call()(hs1, src_w2, dst_w)

    a1 = g1 * lax.rsqrt(rv1 + 1e-5)
    al1 = a1.reshape(1, HID)
    bt1 = (a1 * b1 + be1 - rm1 * a1).reshape(1, HID)
    hs2 = _mid_call(seg1, dinv, al1, bt1, W2)
    seg2 = _agg_call()(hs2, src_w2, dst_w)

    a2 = g2 * lax.rsqrt(rv2 + 1e-5)
    al2 = a2.reshape(1, HID)
    bt2 = (a2 * b2 + be2 - rm2 * a2).reshape(1, HID)
    hs3 = _mid_call(seg2, dinv, al2, bt2, W3)
    seg3 = _agg_call()(hs3, src_w2, dst_w)

    wc2p = jnp.zeros((128, 128), f32).at[:, :NCLS].set(Wc2)
    bc2p = jnp.zeros((1, 128), f32).at[0, :NCLS].set(bc2)
    out = _head_call(seg3, dinv, b3.reshape(1, HID), Wc1, bc1.reshape(1, 128),
                     wc2p, bc2p)
    return out[:NNODES, :NCLS]


# speedup vs baseline: 7.4048x; 1.0237x over previous
import functools

import jax
import jax.numpy as jnp
from jax import lax
from jax.experimental import pallas as pl
from jax.experimental.pallas import tpu as pltpu
from jax.experimental.pallas import tpu_sc as plsc

NNODES = 10000
DIN = 128
HID = 256
NCLS = 20
NEDGE = 320000

NTILE = 16
NPAD = 10240
RPT = NPAD // NTILE
GSZ = 128
GROUPS = 160
EPAD = NTILE * GROUPS * GSZ
GPC = GROUPS // 2
IB = 4
DEG_NB = 8
XGRP = 4

@functools.cache
def _mesh():
    return plsc.VectorSubcoreMesh(core_axis_name="c", subcore_axis_name="s",
                                  num_cores=2, num_subcores=NTILE)


def _deg_body(dst_w, zeros_h, ones_h, pdeg, acc, didx, ones_v, ssem):
    c = lax.axis_index("c")
    s = lax.axis_index("s")
    row0 = s * RPT
    pltpu.sync_copy(zeros_h.at[pl.ds(row0, RPT)], acc.at[pl.ds(row0, RPT)])
    pltpu.sync_copy(ones_h, ones_v)
    pltpu.sync_copy(dst_w.at[s, pl.ds(c * GPC, GPC)], didx)
    plsc.subcore_barrier()

    def outer(i, carry):
        descs = []
        for b in range(DEG_NB):
            g = i * DEG_NB + b
            descs.append(
                pltpu.async_copy(ones_v, acc.at[didx.at[g]], ssem, add=True))
        for d in descs:
            d.wait()
        return carry

    lax.fori_loop(0, GPC // DEG_NB, outer, 0)
    plsc.subcore_barrier()
    pltpu.sync_copy(acc.at[pl.ds(row0, RPT)], pdeg.at[c, pl.ds(row0, RPT)])


@functools.cache
def _deg_call():
    return pl.kernel(
        _deg_body,
        out_type=jax.ShapeDtypeStruct((2, NPAD, 128), jnp.float32),
        mesh=_mesh(),
        scratch_types=[
            pltpu.VMEM_SHARED((NPAD, 128), jnp.float32),
            pltpu.VMEM((GPC, GSZ), jnp.int32),
            pltpu.VMEM((GSZ, 128), jnp.float32),
            pltpu.SemaphoreType.DMA,
        ],
        name="sc_degree_count",
    )


def _agg_body(hs, src_w, dst_w, seg, acc, r0, r1, sa, da, sb, db, isem, g0sem,
              g1sem, s0sem, s1sem):
    c = lax.axis_index("c")
    s = lax.axis_index("s")
    rows = (r0, r1)
    gsems = (g0sem, g1sem)
    ssems = (s0sem, s1sem)
    row0 = s * RPT
    pltpu.sync_copy(hs.at[pl.ds(c * NPAD + row0, RPT)], acc.at[pl.ds(row0, RPT)])
    pltpu.sync_copy(src_w.at[c, s, pl.ds(0, 4)], sa)
    pltpu.sync_copy(dst_w.at[s, pl.ds(0, 4)], da)
    pltpu.async_copy(hs.at[sa.at[0]], rows[0], gsems[0])
    plsc.subcore_barrier()

    def wait_gather(slot):
        pltpu.make_async_copy(hs.at[pl.ds(0, GSZ)], rows[slot],
                              gsems[slot]).wait()

    def drain_scatter(slot):
        pltpu.make_async_copy(hs.at[pl.ds(0, GSZ)], rows[slot],
                              ssems[slot]).wait()

    def body(k, carry):
        base = 8 * k

        def phase(p, s_idx, s_row, d_idx, d_row, prefetches=()):
            sl = p % 2
            nsl = 1 - sl
            if p == 0:
                @pl.when(k > 0)
                def _():
                    drain_scatter(nsl)
            else:
                drain_scatter(nsl)
            for d in prefetches:
                d.wait()
            pltpu.async_copy(hs.at[s_idx.at[s_row]], rows[nsl], gsems[nsl])
            wait_gather(sl)
            return pltpu.async_copy(rows[sl], acc.at[d_idx.at[d_row]],
                                    ssems[sl], add=True)

        phase(0, sa, 1, da, 0)
        pb = (pltpu.async_copy(src_w.at[c, s, pl.ds(base + 4, 4)], sb, isem),
              pltpu.async_copy(dst_w.at[s, pl.ds(base + 4, 4)], db, isem))
        phase(1, sa, 2, da, 1)
        phase(2, sa, 3, da, 2)
        phase(3, sb, 0, da, 3, prefetches=pb)
        phase(4, sb, 1, db, 0)
        pa = (pltpu.async_copy(src_w.at[c, s, pl.ds(base + 8, 4)], sa, isem),
              pltpu.async_copy(dst_w.at[s, pl.ds(base + 8, 4)], da, isem))
        phase(5, sb, 2, db, 1)
        phase(6, sb, 3, db, 2)
        phase(7, sa, 0, db, 3, prefetches=pa)
        return carry

    lax.fori_loop(0, GROUPS // 8, body, 0)
    drain_scatter(1)
    wait_gather(0)
    plsc.subcore_barrier()
    pltpu.sync_copy(acc.at[pl.ds(row0, RPT)], seg.at[c, pl.ds(row0, RPT)])


@functools.cache
def _agg_call():
    return pl.kernel(
        _agg_body,
        out_type=jax.ShapeDtypeStruct((2, NPAD, 128), jnp.float32),
        mesh=_mesh(),
        scratch_types=[
            pltpu.VMEM_SHARED((NPAD, 128), jnp.float32),
            pltpu.VMEM((GSZ, 128), jnp.float32),
            pltpu.VMEM((GSZ, 128), jnp.float32),
            pltpu.VMEM((IB, GSZ), jnp.int32),
            pltpu.VMEM((IB, GSZ), jnp.int32),
            pltpu.VMEM((IB, GSZ), jnp.int32),
            pltpu.VMEM((IB, GSZ), jnp.int32),
            pltpu.SemaphoreType.DMA,
            pltpu.SemaphoreType.DMA,
            pltpu.SemaphoreType.DMA,
            pltpu.SemaphoreType.DMA,
            pltpu.SemaphoreType.DMA,
        ],
        name="sc_edge_aggregate",
    )


RBLK = 512
MGRID = NPAD // RBLK


def _k1_body(x_ref, pd_ref, w_ref, hs_ref, dinv_ref):
    pd = pd_ref[...]
    cnt = pd[0, :, 0:1] + pd[1, :, 0:1]
    dv = lax.rsqrt(1.0 + cnt)
    dinv_ref[...] = dv
    hs_ref[...] = dv * jnp.dot(x_ref[...], w_ref[...],
                               preferred_element_type=jnp.float32)


def _k1_call(xp, pdeg, w1):
    return pl.pallas_call(
        _k1_body,
        grid=(MGRID, 2),
        in_specs=[
            pl.BlockSpec((RBLK, DIN), lambda i, h: (i, 0)),
            pl.BlockSpec((2, RBLK, 128), lambda i, h: (0, i, 0)),
            pl.BlockSpec((DIN, 128), lambda i, h: (0, h)),
        ],
        out_specs=[
            pl.BlockSpec((RBLK, 128), lambda i, h: (h * MGRID + i, 0)),
            pl.BlockSpec((RBLK, 1), lambda i, h: (i, 0)),
        ],
        out_shape=[
            jax.ShapeDtypeStruct((2 * NPAD, 128), jnp.float32),
            jax.ShapeDtypeStruct((NPAD, 1), jnp.float32),
        ],
    )(xp, pdeg, w1)


def _mid_body(seg_ref, dinv_ref, al_ref, be_ref, w_ref, hs_ref):
    sg = seg_ref[...]
    dv = dinv_ref[...]
    z = jnp.concatenate([sg[0], sg[1]], axis=1)
    z = jnp.maximum(dv * z * al_ref[...] + be_ref[...], 0.0)
    hs_ref[...] = dv * jnp.dot(z, w_ref[...],
                               preferred_element_type=jnp.float32)


def _mid_call(seg, dinv, alpha, beta, w):
    return pl.pallas_call(
        _mid_body,
        grid=(MGRID, 2),
        in_specs=[
            pl.BlockSpec((2, RBLK, 128), lambda i, h: (0, i, 0)),
            pl.BlockSpec((RBLK, 1), lambda i, h: (i, 0)),
            pl.BlockSpec((1, HID), lambda i, h: (0, 0)),
            pl.BlockSpec((1, HID), lambda i, h: (0, 0)),
            pl.BlockSpec((HID, 128), lambda i, h: (0, h)),
        ],
        out_specs=pl.BlockSpec((RBLK, 128), lambda i, h: (h * MGRID + i, 0)),
        out_shape=jax.ShapeDtypeStruct((2 * NPAD, 128), jnp.float32),
    )(seg, dinv, alpha, beta, w)


def _head_body(seg_ref, dinv_ref, b3_ref, wc1_ref, bc1_ref, wc2_ref, bc2_ref,
               out_ref):
    sg = seg_ref[...]
    dv = dinv_ref[...]
    z3 = dv * jnp.concatenate([sg[0], sg[1]], axis=1) + b3_ref[...]
    t = jnp.maximum(
        jnp.dot(z3, wc1_ref[...], preferred_element_type=jnp.float32)
        + bc1_ref[...], 0.0)
    out_ref[...] = (jnp.dot(t, wc2_ref[...],
                            preferred_element_type=jnp.float32) + bc2_ref[...])


def _head_call(seg, dinv, b3, wc1, bc1, wc2p, bc2p):
    return pl.pallas_call(
        _head_body,
        grid=(MGRID,),
        in_specs=[
            pl.BlockSpec((2, RBLK, 128), lambda i: (0, i, 0)),
            pl.BlockSpec((RBLK, 1), lambda i: (i, 0)),
            pl.BlockSpec((1, HID), lambda i: (0, 0)),
            pl.BlockSpec((HID, 128), lambda i: (0, 0)),
            pl.BlockSpec((1, 128), lambda i: (0, 0)),
            pl.BlockSpec((128, 128), lambda i: (0, 0)),
            pl.BlockSpec((1, 128), lambda i: (0, 0)),
        ],
        out_specs=pl.BlockSpec((RBLK, 128), lambda i: (i, 0)),
        out_shape=jax.ShapeDtypeStruct((NPAD, 128), jnp.float32),
    )(seg, dinv, b3, wc1, bc1, wc2p, bc2p)


def kernel(x, edge_index, W1, b1, g1, be1, rm1, rv1, W2, b2, g2, be2, rm2,
           rv2, W3, b3, Wc1, bc1, Wc2, bc2):
    f32 = jnp.float32
    src = edge_index[0]
    dst = edge_index[1]

    xp = jnp.concatenate([x, jnp.zeros((NPAD - NNODES, DIN), f32)], axis=0)
    pad = jnp.full((EPAD - NEDGE,), NPAD - 1, jnp.int32)
    xpad = jnp.full((NTILE, XGRP, GSZ), NPAD - 1, jnp.int32)
    src_w = jnp.concatenate(
        [jnp.concatenate([src, pad]).reshape(NTILE, GROUPS, GSZ), xpad], axis=1)
    dst_w = jnp.concatenate(
        [jnp.concatenate([dst, pad]).reshape(NTILE, GROUPS, GSZ), xpad], axis=1)
    src_w2 = jnp.stack([src_w, src_w + NPAD])

    zeros_h = jnp.zeros((NPAD, 128), f32)
    ones_h = jnp.ones((GSZ, 128), f32)
    pdeg = _deg_call()(dst_w, zeros_h, ones_h)

    hs1, dinv = _k1_call(xp, pdeg, W1)
    seg1 = _agg_call()(hs1, src_w2, dst_w)

    a1 = g1 * lax.rsqrt(rv1 + 1e-5)
    al1 = a1.reshape(1, HID)
    bt1 = (a1 * b1 + be1 - rm1 * a1).reshape(1, HID)
    hs2 = _mid_call(seg1, dinv, al1, bt1, W2)
    seg2 = _agg_call()(hs2, src_w2, dst_w)

    a2 = g2 * lax.rsqrt(rv2 + 1e-5)
    al2 = a2.reshape(1, HID)
    bt2 = (a2 * b2 + be2 - rm2 * a2).reshape(1, HID)
    hs3 = _mid_call(seg2, dinv, al2, bt2, W3)
    seg3 = _agg_call()(hs3, src_w2, dst_w)

    wc2p = jnp.zeros((128, 128), f32).at[:, :NCLS].set(Wc2)
    bc2p = jnp.zeros((1, 128), f32).at[0, :NCLS].set(bc2)
    out = _head_call(seg3, dinv, b3.reshape(1, HID), Wc1, bc1.reshape(1, 128),
                     wc2p, bc2p)
    return out[:NNODES, :NCLS]
